# baseline (speedup 1.0000x reference)
import functools

import jax
import jax.numpy as jnp
from jax import lax
from jax.experimental import pallas as pl
from jax.experimental.pallas import tpu as pltpu
from jax.experimental.pallas import tpu_sc as plsc

F = 26
V = 100000
D = 32
B = 16384
CD = 64
XR = CD // D
OR = F + XR

NC = 2
NS = 16
NW = NC * NS
BW = B // NW
CB = 64
NCHUNK = BW // CB
ROWS = CB * F
NT = ROWS // 128
NIV = ROWS // 16


def _body(x2, vals, tab, out, vals_v, gidx_v, oidx_v, xidx_v, rows_v, x_v,
          patg_v, pato_v, patx_v, sg0, sg1, ss0, ss1):
    wid = lax.axis_index("s") * NC + lax.axis_index("c")
    sem_g = (sg0, sg1)
    sem_s = (ss0, ss1)

    idx16 = lax.iota(jnp.int32, 16)

    def _splat(c):
        return jnp.full((16,), c, jnp.int32)

    for j in range(13):
        q = idx16 + j * 16
        f = lax.rem(q, _splat(F))
        patg_v[j, :] = f * V
        pato_v[j, :] = lax.div(q, _splat(F)) * OR + f
    patx_v[:] = lax.div(idx16, _splat(XR)) * OR + F + lax.rem(idx16, _splat(XR))

    def load_and_index(c, p):
        r0 = wid * BW + c * CB
        pltpu.sync_copy(vals.at[pl.ds(r0 * F, ROWS)], vals_v.at[p])
        pltpu.sync_copy(x2.at[pl.ds(r0 * XR, CB * XR)], x_v.at[p])
        base_out = r0 * OR

        def ivec(i, carry):
            j = lax.rem(i, 13)
            g = lax.div(i, 13)
            vv = vals_v[p, pl.ds(i * 16, 16)]
            gidx_v[p, pl.ds(i * 16, 16)] = vv + patg_v[j, :]
            goff = base_out + g * (208 // F * OR)
            oidx_v[p, pl.ds(i * 16, 16)] = pato_v[j, :] + goff
            return carry

        lax.fori_loop(0, NIV, ivec, 0)
        for k in range(CB * XR // 16):
            xidx_v[p, pl.ds(k * 16, 16)] = patx_v[:] + (base_out + k * 8 * OR)

    def fire_gathers(p):
        return [
            pltpu.async_copy(tab.at[gidx_v.at[p]], rows_v.at[p], sem_g[p])
        ]

    def fire_scatters(p):
        return [
            pltpu.async_copy(rows_v.at[p], out.at[oidx_v.at[p]], sem_s[p]),
            pltpu.async_copy(x_v.at[p], out.at[xidx_v.at[p]], sem_s[p]),
        ]

    g_cp = {}
    s_cp = {}
    for c in range(NCHUNK):
        p = c & 1
        if c >= 2:
            for cp in s_cp[c - 2]:
                cp.wait()
        load_and_index(c, p)
        g_cp[c] = fire_gathers(p)
        if c >= 1:
            for cp in g_cp[c - 1]:
                cp.wait()
            s_cp[c - 1] = fire_scatters(1 - p)
    for cp in s_cp[NCHUNK - 2]:
        cp.wait()
    last = NCHUNK - 1
    for cp in g_cp[last]:
        cp.wait()
    for cp in fire_scatters(last & 1):
        cp.wait()


@jax.jit
def _run(x2, vals, tab):
    mesh = plsc.VectorSubcoreMesh(core_axis_name="c", subcore_axis_name="s")
    kern = functools.partial(
        pl.kernel,
        out_type=jax.ShapeDtypeStruct((B * OR, D), jnp.float32),
        mesh=mesh,
        compiler_params=pltpu.CompilerParams(use_tc_tiling_on_sc=False),
        scratch_types=[
            pltpu.VMEM((2, ROWS), jnp.int32),
            pltpu.VMEM((2, ROWS), jnp.int32),
            pltpu.VMEM((2, ROWS), jnp.int32),
            pltpu.VMEM((2, 128), jnp.int32),
            pltpu.VMEM((2, ROWS, D), jnp.float32),
            pltpu.VMEM((2, CB * XR, D), jnp.float32),
            pltpu.VMEM((13, 16), jnp.int32),
            pltpu.VMEM((13, 16), jnp.int32),
            pltpu.VMEM((16,), jnp.int32),
            pltpu.SemaphoreType.DMA,
            pltpu.SemaphoreType.DMA,
            pltpu.SemaphoreType.DMA,
            pltpu.SemaphoreType.DMA,
        ],
    )(_body)
    return kern(x2, vals, tab)


def kernel(x, values, tables):
    x2 = x.reshape(B * XR, D)
    vals = values.reshape(B * F)
    tab = tables.reshape(F * V, D)
    out = _run(x2, vals, tab)
    return out.reshape(B, F * D + CD)


# speedup vs baseline: 2.6074x; 2.6074x over previous
import functools

import jax
import jax.numpy as jnp
from jax import lax
from jax.experimental import pallas as pl
from jax.experimental.pallas import tpu as pltpu
from jax.experimental.pallas import tpu_sc as plsc

F = 26
V = 100000
D = 32
B = 16384
CD = 64

NC = 2
NS = 16
NW = NC * NS
TR = F * D
RPW = TR // NW
XRW = CD // NW
OCH = 4096
NV = B // 16


def _body(xT, valsT, tabR, outT, row_v, idx_v, out_v):
    wid = lax.axis_index("s") * NC + lax.axis_index("c")

    def do_row(j, carry):
        r = wid * RPW + j
        f = lax.div(r, D)
        pltpu.sync_copy(tabR.at[r], row_v)
        pltpu.sync_copy(valsT.at[f], idx_v)

        def chunk(q, c2):
            def gath(i, c3):
                vv = idx_v[pl.ds(q * OCH + i * 16, 16)]
                out_v[pl.ds(i * 16, 16)] = plsc.load_gather(row_v, [vv])
                return c3

            lax.fori_loop(0, OCH // 16, gath, 0)
            pltpu.sync_copy(out_v, outT.at[r, pl.ds(q * OCH, OCH)])
            return c2

        lax.fori_loop(0, B // OCH, chunk, 0)
        return carry

    lax.fori_loop(0, RPW, do_row, 0)

    def do_xrow(k, carry):
        xr = wid * XRW + k

        def xchunk(q, c2):
            pltpu.sync_copy(xT.at[xr, pl.ds(q * OCH, OCH)], out_v)
            pltpu.sync_copy(out_v, outT.at[TR + xr, pl.ds(q * OCH, OCH)])
            return c2

        lax.fori_loop(0, B // OCH, xchunk, 0)
        return carry

    lax.fori_loop(0, XRW, do_xrow, 0)


@jax.jit
def _run(xT, valsT, tabR):
    mesh = plsc.VectorSubcoreMesh(core_axis_name="c", subcore_axis_name="s")
    kern = functools.partial(
        pl.kernel,
        out_type=jax.ShapeDtypeStruct((TR + CD, B), jnp.float32),
        mesh=mesh,
        compiler_params=pltpu.CompilerParams(needs_layout_passes=False),
        scratch_types=[
            pltpu.VMEM((V,), jnp.float32),
            pltpu.VMEM((B,), jnp.int32),
            pltpu.VMEM((OCH,), jnp.float32),
        ],
    )(_body)
    return kern(xT, valsT, tabR)


def kernel(x, values, tables):
    tabR = jnp.swapaxes(tables, 1, 2).reshape(TR, V)
    valsT = values.T
    xT = x.T
    outT = _run(xT, valsT, tabR)
    return outT.T


# speedup vs baseline: 5.0479x; 1.9360x over previous
import functools

import jax
import jax.numpy as jnp
from jax import lax
from jax.experimental import pallas as pl
from jax.experimental.pallas import tpu as pltpu
from jax.experimental.pallas import tpu_sc as plsc

F = 26
V = 100000
D = 32
B = 16384
CD = 64

NC = 2
NS = 16
NW = NC * NS
TR = F * D
RPW = TR // NW
XRW = CD // NW
OCH = 4096
NV = B // 16


def _body(xT, valsT, tabR, outT, row_v, idx_v, out_v):
    wid = lax.axis_index("s") * NC + lax.axis_index("c")

    def do_row(j, prev_f):
        r = wid * RPW + j
        f = lax.div(r, D)

        @pl.when(f != prev_f)
        def _():
            pltpu.sync_copy(valsT.at[f], idx_v)

        pltpu.sync_copy(tabR.at[r], row_v)

        def chunk(q, c2):
            @plsc.parallel_loop(0, OCH // 16, 1, unroll=8)
            def gath(i):
                vv = idx_v[pl.ds(q * OCH + i * 16, 16)]
                out_v[pl.ds(i * 16, 16)] = plsc.load_gather(row_v, [vv])

            pltpu.sync_copy(out_v, outT.at[r, pl.ds(q * OCH, OCH)])
            return c2

        lax.fori_loop(0, B // OCH, chunk, 0)
        return f

    lax.fori_loop(0, RPW, do_row, jnp.int32(-1))

    def do_xrow(k, carry):
        xr = wid * XRW + k

        def xchunk(q, c2):
            pltpu.sync_copy(xT.at[xr, pl.ds(q * OCH, OCH)], out_v)
            pltpu.sync_copy(out_v, outT.at[TR + xr, pl.ds(q * OCH, OCH)])
            return c2

        lax.fori_loop(0, B // OCH, xchunk, 0)
        return carry

    lax.fori_loop(0, XRW, do_xrow, 0)


@jax.jit
def _run(xT, valsT, tabR):
    mesh = plsc.VectorSubcoreMesh(core_axis_name="c", subcore_axis_name="s")
    kern = functools.partial(
        pl.kernel,
        out_type=jax.ShapeDtypeStruct((TR + CD, B), jnp.float32),
        mesh=mesh,
        compiler_params=pltpu.CompilerParams(needs_layout_passes=False),
        scratch_types=[
            pltpu.VMEM((V,), jnp.float32),
            pltpu.VMEM((B,), jnp.int32),
            pltpu.VMEM((OCH,), jnp.float32),
        ],
    )(_body)
    return kern(xT, valsT, tabR)


def kernel(x, values, tables):
    tabR = jnp.swapaxes(tables, 1, 2).reshape(TR, V)
    valsT = values.T
    xT = x.T
    outT = _run(xT, valsT, tabR)
    return outT.T


# speedup vs baseline: 5.1053x; 1.0114x over previous
import functools

import jax
import jax.numpy as jnp
from jax import lax
from jax.experimental import pallas as pl
from jax.experimental.pallas import tpu as pltpu
from jax.experimental.pallas import tpu_sc as plsc

F = 26
V = 100000
D = 32
B = 16384
CD = 64

NC = 2
NS = 16
NW = NC * NS
TR = F * D
RPW = TR // NW
XRW = CD // NW
OCH = 4096
NV = B // 16


def _body(xT, valsT, tabR, outT, row_v, idx_v, out_v, sem_r, sem_o):
    wid = lax.axis_index("s") * NC + lax.axis_index("c")

    def do_row(j, prev_f):
        r = wid * RPW + j
        f = lax.div(r, D)
        rcp = pltpu.async_copy(tabR.at[r], row_v, sem_r)

        @pl.when(f != prev_f)
        def _():
            pltpu.sync_copy(valsT.at[f], idx_v)

        rcp.wait()
        cps = [None, None]
        for q in range(B // OCH):
            s = q % 2
            if cps[s] is not None:
                cps[s].wait()

            @plsc.parallel_loop(0, OCH // 16, 1, unroll=8)
            def gath(i):
                vv = idx_v[pl.ds(q * OCH + i * 16, 16)]
                out_v[s, pl.ds(i * 16, 16)] = plsc.load_gather(row_v, [vv])

            cps[s] = pltpu.async_copy(
                out_v.at[s], outT.at[r, pl.ds(q * OCH, OCH)], sem_o
            )
        for cp in cps:
            cp.wait()
        return f

    lax.fori_loop(0, RPW, do_row, jnp.int32(-1))

    def do_xrow(k, carry):
        xr = wid * XRW + k

        def xchunk(q, c2):
            pltpu.sync_copy(xT.at[xr, pl.ds(q * OCH, OCH)], out_v.at[0])
            pltpu.sync_copy(out_v.at[0], outT.at[TR + xr, pl.ds(q * OCH, OCH)])
            return c2

        lax.fori_loop(0, B // OCH, xchunk, 0)
        return carry

    lax.fori_loop(0, XRW, do_xrow, 0)


@jax.jit
def _run(xT, valsT, tabR):
    mesh = plsc.VectorSubcoreMesh(core_axis_name="c", subcore_axis_name="s")
    kern = functools.partial(
        pl.kernel,
        out_type=jax.ShapeDtypeStruct((TR + CD, B), jnp.float32),
        mesh=mesh,
        compiler_params=pltpu.CompilerParams(needs_layout_passes=False),
        scratch_types=[
            pltpu.VMEM((V,), jnp.float32),
            pltpu.VMEM((B,), jnp.int32),
            pltpu.VMEM((2, OCH), jnp.float32),
            pltpu.SemaphoreType.DMA,
            pltpu.SemaphoreType.DMA,
        ],
    )(_body)
    return kern(xT, valsT, tabR)


def kernel(x, values, tables):
    tabR = jnp.swapaxes(tables, 1, 2).reshape(TR, V)
    valsT = values.T
    xT = x.T
    outT = _run(xT, valsT, tabR)
    return outT.T


# speedup vs baseline: 5.1166x; 1.0022x over previous
import functools

import jax
import jax.numpy as jnp
from jax import lax
from jax.experimental import pallas as pl
from jax.experimental.pallas import tpu as pltpu
from jax.experimental.pallas import tpu_sc as plsc

F = 26
V = 100000
D = 32
B = 16384
CD = 64

NC = 2
NS = 16
NW = NC * NS
TR = F * D
RPW = TR // NW
XRW = CD // NW
OCH = 4096
NV = B // 16


def _body(xT, valsT, tabR, outT, row_v, idx_v, out_v, sem_r, sem_o):
    wid = lax.axis_index("s") * NC + lax.axis_index("c")

    def do_row(j, prev_f):
        r = wid * RPW + j
        f = lax.div(r, D)
        rcps = [pltpu.async_copy(tabR.at[r], row_v, sem_r)]

        @pl.when(f != prev_f)
        def _():
            pltpu.sync_copy(valsT.at[f], idx_v)

        for rcp in rcps:
            rcp.wait()
        cps = [None, None]
        for q in range(B // OCH):
            s = q % 2
            if cps[s] is not None:
                cps[s].wait()

            @plsc.parallel_loop(0, OCH // 16, 1, unroll=16)
            def gath(i):
                vv = idx_v[pl.ds(q * OCH + i * 16, 16)]
                out_v[s, pl.ds(i * 16, 16)] = plsc.load_gather(row_v, [vv])

            cps[s] = pltpu.async_copy(
                out_v.at[s], outT.at[r, pl.ds(q * OCH, OCH)], sem_o
            )
        for cp in cps:
            cp.wait()
        return f

    lax.fori_loop(0, RPW, do_row, jnp.int32(-1))

    def do_xrow(k, carry):
        xr = wid * XRW + k

        def xchunk(q, c2):
            pltpu.sync_copy(xT.at[xr, pl.ds(q * OCH, OCH)], out_v.at[0])
            pltpu.sync_copy(out_v.at[0], outT.at[TR + xr, pl.ds(q * OCH, OCH)])
            return c2

        lax.fori_loop(0, B // OCH, xchunk, 0)
        return carry

    lax.fori_loop(0, XRW, do_xrow, 0)


@jax.jit
def _run(xT, valsT, tabR):
    mesh = plsc.VectorSubcoreMesh(core_axis_name="c", subcore_axis_name="s")
    kern = functools.partial(
        pl.kernel,
        out_type=jax.ShapeDtypeStruct((TR + CD, B), jnp.float32),
        mesh=mesh,
        compiler_params=pltpu.CompilerParams(needs_layout_passes=False),
        scratch_types=[
            pltpu.VMEM((V,), jnp.float32),
            pltpu.VMEM((B,), jnp.int32),
            pltpu.VMEM((2, OCH), jnp.float32),
            pltpu.SemaphoreType.DMA,
            pltpu.SemaphoreType.DMA,
        ],
    )(_body)
    return kern(xT, valsT, tabR)


def kernel(x, values, tables):
    tabR = jnp.swapaxes(tables, 1, 2).reshape(TR, V)
    valsT = values.T
    xT = x.T
    outT = _run(xT, valsT, tabR)
    return outT.T


# speedup vs baseline: 5.2865x; 1.0332x over previous
import functools

import jax
import jax.numpy as jnp
from jax import lax
from jax.experimental import pallas as pl
from jax.experimental.pallas import tpu as pltpu
from jax.experimental.pallas import tpu_sc as plsc

F = 26
V = 100000
D = 32
B = 16384
CD = 64

NC = 2
NS = 16
NW = NC * NS
TR = F * D
RPW = TR // NW
XRW = CD // NW
OCH = 4096
NV = B // 16


def _body(xT, valsT, tabR, outT, row_v, idx_v, out_v, sem_r, sem_o, sem_x):
    wid = lax.axis_index("s") * NC + lax.axis_index("c")

    xcps = [
        pltpu.async_copy(
            xT.at[wid * XRW + k], outT.at[TR + wid * XRW + k], sem_x
        )
        for k in range(XRW)
    ]

    def do_row(j, prev_f):
        r = wid * RPW + j
        f = lax.div(r, D)
        rcps = [pltpu.async_copy(tabR.at[r], row_v, sem_r)]

        @pl.when(f != prev_f)
        def _():
            pltpu.sync_copy(valsT.at[f], idx_v)

        for rcp in rcps:
            rcp.wait()
        cps = [None, None]
        for q in range(B // OCH):
            s = q % 2
            if cps[s] is not None:
                cps[s].wait()

            @plsc.parallel_loop(0, OCH // 16, 1, unroll=16)
            def gath(i):
                vv = idx_v[pl.ds(q * OCH + i * 16, 16)]
                out_v[s, pl.ds(i * 16, 16)] = plsc.load_gather(row_v, [vv])

            cps[s] = pltpu.async_copy(
                out_v.at[s], outT.at[r, pl.ds(q * OCH, OCH)], sem_o
            )
        for cp in cps:
            cp.wait()
        return f

    lax.fori_loop(0, RPW, do_row, jnp.int32(-1))
    for cp in xcps:
        cp.wait()


@jax.jit
def _run(xT, valsT, tabR):
    mesh = plsc.VectorSubcoreMesh(core_axis_name="c", subcore_axis_name="s")
    kern = functools.partial(
        pl.kernel,
        out_type=jax.ShapeDtypeStruct((TR + CD, B), jnp.float32),
        mesh=mesh,
        compiler_params=pltpu.CompilerParams(needs_layout_passes=False),
        scratch_types=[
            pltpu.VMEM((V,), jnp.float32),
            pltpu.VMEM((B,), jnp.int32),
            pltpu.VMEM((2, OCH), jnp.float32),
            pltpu.SemaphoreType.DMA,
            pltpu.SemaphoreType.DMA,
            pltpu.SemaphoreType.DMA,
        ],
    )(_body)
    return kern(xT, valsT, tabR)


def kernel(x, values, tables):
    tabR = jnp.swapaxes(tables, 1, 2).reshape(TR, V)
    valsT = values.T
    xT = x.T
    outT = _run(xT, valsT, tabR)
    return outT.T
